# baseline (speedup 1.0000x reference)
import functools

import jax
import jax.numpy as jnp
from jax import lax
from jax.experimental import pallas as pl
from jax.experimental.pallas import tpu as pltpu
from jax.experimental.pallas import tpu_sc as plsc

T = 2048
D = 768
E = 8
K = 2
H = 3072

M = 192
HC = 1536
NP = 5760
NT = NP // M
NH = H // HC


def _router_body(x_ref, wr_ref, br_ref, e0_ref, e1_ref, p0_ref, p1_ref):
    logits = lax.dot_general(
        x_ref[...], wr_ref[...], (((1,), (1,)), ((), ())),
        preferred_element_type=jnp.float32,
    ) + br_ref[...][None, :]
    iota_e = lax.broadcasted_iota(jnp.int32, (T, E), 1)
    m0 = jnp.max(logits, axis=1, keepdims=True)
    i0 = jnp.min(jnp.where(logits == m0, iota_e, E), axis=1, keepdims=True)
    masked = jnp.where(iota_e == i0, -jnp.inf, logits)
    m1 = jnp.max(masked, axis=1, keepdims=True)
    i1 = jnp.min(jnp.where(masked == m1, iota_e, E), axis=1, keepdims=True)
    p0 = 1.0 / (1.0 + jnp.exp(m1 - m0))
    e0_ref[...] = i0
    e1_ref[...] = i1
    p0_ref[...] = p0
    p1_ref[...] = 1.0 - p0


def _router(x, Wr, br):
    e0, e1, p0, p1 = pl.pallas_call(
        _router_body,
        out_shape=(
            jax.ShapeDtypeStruct((T, 1), jnp.int32),
            jax.ShapeDtypeStruct((T, 1), jnp.int32),
            jax.ShapeDtypeStruct((T, 1), jnp.float32),
            jax.ShapeDtypeStruct((T, 1), jnp.float32),
        ),
    )(x, Wr, br)
    return e0[:, 0], e1[:, 0], p0[:, 0], p1[:, 0]


def _gmm_body(te_ref, xs_ref, wg_ref, bg_ref, wu_ref, bu_ref, wd_ref, bd_ref,
              y_ref, acc_ref):
    h = pl.program_id(0)
    t = pl.program_id(1)

    @pl.when(te_ref[t] >= 0)
    def _():
        xb = xs_ref[...].astype(jnp.bfloat16)
        g = lax.dot_general(xb, wg_ref[0].astype(jnp.bfloat16),
                            (((1,), (1,)), ((), ())),
                            preferred_element_type=jnp.float32)
        g = g + bg_ref[0]
        g = g * jax.nn.sigmoid(g)
        u = lax.dot_general(xb, wu_ref[0].astype(jnp.bfloat16),
                            (((1,), (1,)), ((), ())),
                            preferred_element_type=jnp.float32)
        u = u + bu_ref[0]
        contrib = lax.dot_general((u * g).astype(jnp.bfloat16),
                                  wd_ref[0].astype(jnp.bfloat16),
                                  (((1,), (1,)), ((), ())),
                                  preferred_element_type=jnp.float32)
        rows = pl.ds(t * M, M)

        @pl.when(h == 0)
        def _():
            acc_ref[rows, :] = contrib + bd_ref[0]

        @pl.when((h > 0) & (h < NH - 1))
        def _():
            acc_ref[rows, :] = acc_ref[rows, :] + contrib

        @pl.when(h == NH - 1)
        def _():
            y_ref[...] = acc_ref[rows, :] + contrib


def _gmm(xs, te, Wg, bg, Wu, bu, Wd, bd):
    def we(te, t):
        return jnp.where(te[t] < 0, E - 1, te[t])

    grid_spec = pltpu.PrefetchScalarGridSpec(
        num_scalar_prefetch=1,
        grid=(NH, NT),
        in_specs=[
            pl.BlockSpec((M, D), lambda h, t, te: (t, 0)),
            pl.BlockSpec((1, HC, D), lambda h, t, te: (we(te, t), h, 0)),
            pl.BlockSpec((1, 1, HC),
                         lambda h, t, te: (we(te, t) * NH + h, 0, 0)),
            pl.BlockSpec((1, HC, D), lambda h, t, te: (we(te, t), h, 0)),
            pl.BlockSpec((1, 1, HC),
                         lambda h, t, te: (we(te, t) * NH + h, 0, 0)),
            pl.BlockSpec((1, D, HC), lambda h, t, te: (we(te, t), 0, h)),
            pl.BlockSpec((1, 1, D), lambda h, t, te: (we(te, t), 0, 0)),
        ],
        out_specs=pl.BlockSpec((M, D), lambda h, t, te: (t, 0)),
        scratch_shapes=[pltpu.VMEM((NP, D), jnp.float32)],
    )
    return pl.pallas_call(
        _gmm_body,
        grid_spec=grid_spec,
        out_shape=jax.ShapeDtypeStruct((NP, D), jnp.float32),
        compiler_params=pltpu.CompilerParams(
            dimension_semantics=("arbitrary", "arbitrary"),
        ),
    )(te, xs, Wg, bg.reshape(E * NH, 1, HC), Wu, bu.reshape(E * NH, 1, HC),
      Wd, bd.reshape(E, 1, D))


NW = 16
A = (2 * T) // NW
GR = 64


def _dispatch_body(e0_hbm, e1_hbm, x_hbm,
                   xs_hbm, pos0_hbm, pos1_hbm, te_hbm, cnts_hbm,
                   ea, dst, cnt1, allcnt, te_s, tid, did, rows, sem):
    cid = lax.axis_index("c")
    sid = lax.axis_index("s")
    iota16 = lax.broadcasted_iota(jnp.int32, (16,), 0)
    NC = A // 16

    @pl.when(cid == 0)
    def _():
        @pl.when(sid < NW // 2)
        def _():
            pltpu.sync_copy(e0_hbm.at[pl.ds(sid * A, A)], ea)

        @pl.when(sid >= NW // 2)
        def _():
            pltpu.sync_copy(e1_hbm.at[pl.ds((sid - NW // 2) * A, A)], ea)

        dn = lax.GatherDimensionNumbers(offset_dims=(),
                                        collapsed_slice_dims=(0,),
                                        start_index_map=(0,))

        def gat(v, idx):
            return lax.gather(v, idx[:, None], dn, (1,),
                              mode=lax.GatherScatterMode.PROMISE_IN_BOUNDS)

        rot_idx = [((iota16 - k) + 16) & 15 for k in (1, 2, 4, 8)]
        sh_idx = [jnp.maximum(iota16 - k, 0) for k in (1, 2, 4, 8)]
        sh_msk = [iota16 >= k for k in (1, 2, 4, 8)]

        def allsum(v):
            for idx in rot_idx:
                v = v + gat(v, idx)
            return v

        def iprefix(v):
            for idx, mk in zip(sh_idx, sh_msk):
                v = v + jnp.where(mk, gat(v, idx), 0)
            return v

        cv = jnp.zeros((16,), jnp.int32)
        for c in range(NC):
            ch = ea[pl.ds(c * 16, 16)]
            for e in range(E):
                pop = allsum(jnp.where(ch == e, 1, 0))
                cv = cv + jnp.where(iota16 == e, pop, 0)
        cnt1[...] = cv
        pltpu.sync_copy(cnt1, cnts_hbm.at[sid])
        plsc.subcore_barrier()
        pltpu.sync_copy(cnts_hbm, allcnt)

        tot = jnp.zeros((16,), jnp.int32)
        for wp in range(NW):
            tot = tot + allcnt[wp, :]
        a = tot + (M - 1)
        pv = (((a >> 6) * 43691) >> 17) * M
        csum = iprefix(pv)
        basev = csum - pv
        sv = basev
        for wp in range(NW):
            sv = sv + jnp.where(wp < sid, allcnt[wp, :], 0)

        for c in range(NC):
            ch = ea[pl.ds(c * 16, 16)]
            dv = gat(sv, ch)
            inc = jnp.zeros((16,), jnp.int32)
            for e in range(E):
                m = ch == e
                ip = iprefix(jnp.where(m, 1, 0))
                dv = dv + jnp.where(m, ip - 1, 0)
                cnt_e = jnp.full((16,), ip[15], jnp.int32)
                inc = inc + jnp.where(iota16 == e, cnt_e, 0)
            dst[pl.ds(c * 16, 16)] = dv
            sv = sv + inc

        @pl.when(sid < NW // 2)
        def _():
            pltpu.sync_copy(dst, pos0_hbm.at[pl.ds(sid * A, A)])

        @pl.when(sid >= NW // 2)
        def _():
            pltpu.sync_copy(dst, pos1_hbm.at[pl.ds((sid - NW // 2) * A, A)])

        tbase = jnp.where(sid < NW // 2, sid, sid - NW // 2) * A
        for r in range(A // GR):
            for v in range(GR // 16):
                tid[pl.ds(v * 16, 16)] = iota16 + (tbase + r * GR + v * 16)
                did[pl.ds(v * 16, 16)] = dst[pl.ds(r * GR + v * 16, 16)]
            pltpu.async_copy(x_hbm.at[tid], rows, sem).wait()
            pltpu.async_copy(rows, xs_hbm.at[did], sem).wait()

        @pl.when(sid == 0)
        def _():
            used = jnp.full((16,), basev[E], jnp.int32)
            for ch2 in range(2):
                bm = (iota16 + 16 * ch2) * M
                cntge = jnp.zeros((16,), jnp.int32)
                for e in range(E):
                    be = jnp.full((16,), basev[e], jnp.int32)
                    cntge = cntge + jnp.where(bm >= be, 1, 0)
                te_s[pl.ds(ch2 * 16, 16)] = jnp.where(bm >= used, -1,
                                                      cntge - 1)
            pltpu.sync_copy(te_s, te_hbm)


def _dispatch(e0, e1, x):
    mesh = plsc.VectorSubcoreMesh(core_axis_name="c", subcore_axis_name="s")
    f = functools.partial(
        pl.kernel,
        mesh=mesh,
        out_type=(
            jax.ShapeDtypeStruct((NP, D), jnp.float32),
            jax.ShapeDtypeStruct((T,), jnp.int32),
            jax.ShapeDtypeStruct((T,), jnp.int32),
            jax.ShapeDtypeStruct((32,), jnp.int32),
            jax.ShapeDtypeStruct((NW, 16), jnp.int32),
        ),
        scratch_types=[
            pltpu.VMEM((A,), jnp.int32),
            pltpu.VMEM((A,), jnp.int32),
            pltpu.VMEM((16,), jnp.int32),
            pltpu.VMEM((NW, 16), jnp.int32),
            pltpu.VMEM((32,), jnp.int32),
            pltpu.VMEM((GR,), jnp.int32),
            pltpu.VMEM((GR,), jnp.int32),
            pltpu.VMEM((GR, D), jnp.float32),
            pltpu.SemaphoreType.DMA,
        ],
    )(_dispatch_body)
    return f(e0, e1, x)


TW = T // 32


def _combine_body(ys_hbm, pos0_hbm, pos1_hbm, p0_hbm, p1_hbm, out_hbm,
                  i0b, i1b, p0b, p1b, r0, r1, sem):
    cid = lax.axis_index("c")
    sid = lax.axis_index("s")
    base = (cid * 16 + sid) * TW
    pltpu.sync_copy(pos0_hbm.at[pl.ds(base, TW)], i0b)
    pltpu.sync_copy(pos1_hbm.at[pl.ds(base, TW)], i1b)
    pltpu.sync_copy(p0_hbm.at[pl.ds(base, TW)], p0b.at[pl.ds(0, TW)])
    pltpu.sync_copy(p1_hbm.at[pl.ds(base, TW)], p1b.at[pl.ds(0, TW)])
    c0 = pltpu.async_copy(ys_hbm.at[i0b], r0, sem)
    c1 = pltpu.async_copy(ys_hbm.at[i1b], r1, sem)
    c0.wait()
    c1.wait()

    def tok(i, c):
        a0 = p0b[pl.ds(i, 16)][0]
        a1 = p1b[pl.ds(i, 16)][0]
        for v in range(D // 16):
            sl = pl.ds(v * 16, 16)
            r0[i, sl] = a0 * r0[i, sl] + a1 * r1[i, sl]
        return c

    lax.fori_loop(0, TW, tok, 0)
    pltpu.sync_copy(r0, out_hbm.at[pl.ds(base, TW), :])


def _combine(ys, pos0, pos1, p0, p1):
    mesh = plsc.VectorSubcoreMesh(core_axis_name="c", subcore_axis_name="s")
    f = functools.partial(
        pl.kernel,
        mesh=mesh,
        out_type=jax.ShapeDtypeStruct((T, D), jnp.float32),
        scratch_types=[
            pltpu.VMEM((TW,), jnp.int32),
            pltpu.VMEM((TW,), jnp.int32),
            pltpu.VMEM((TW + 16,), jnp.float32),
            pltpu.VMEM((TW + 16,), jnp.float32),
            pltpu.VMEM((TW, D), jnp.float32),
            pltpu.VMEM((TW, D), jnp.float32),
            pltpu.SemaphoreType.DMA,
        ],
    )(_combine_body)
    return f(ys, pos0, pos1, p0, p1)


def kernel(x, Wr, br, Wg, bg, Wu, bu, Wd, bd):
    e0, e1, p0, p1 = _router(x, Wr, br)
    xs, pos0, pos1, te, _ = _dispatch(e0, e1, x)
    ys = _gmm(xs, te, Wg, bg, Wu, bu, Wd, bd)
    return _combine(ys, pos0, pos1, p0, p1)


# speedup vs baseline: 1.0257x; 1.0257x over previous
import functools

import jax
import jax.numpy as jnp
from jax import lax
from jax.experimental import pallas as pl
from jax.experimental.pallas import tpu as pltpu
from jax.experimental.pallas import tpu_sc as plsc

T = 2048
D = 768
E = 8
K = 2
H = 3072

M = 192
HC = 1536
NP = 5760
NT = NP // M
NH = H // HC


def _router_body(x_ref, wr_ref, br_ref, e0_ref, e1_ref, p0_ref, p1_ref):
    lg = lax.dot_general(
        wr_ref[...], x_ref[...], (((1,), (1,)), ((), ())),
        preferred_element_type=jnp.float32,
    ) + br_ref[...][:, None]
    iota_e = lax.broadcasted_iota(jnp.int32, (E, T), 0)
    m0 = jnp.max(lg, axis=0, keepdims=True)
    i0 = jnp.min(jnp.where(lg == m0, iota_e, E), axis=0, keepdims=True)
    masked = jnp.where(iota_e == i0, -jnp.inf, lg)
    m1 = jnp.max(masked, axis=0, keepdims=True)
    i1 = jnp.min(jnp.where(masked == m1, iota_e, E), axis=0, keepdims=True)
    p0 = 1.0 / (1.0 + jnp.exp(m1 - m0))
    e0_ref[...] = i0
    e1_ref[...] = i1
    p0_ref[...] = p0
    p1_ref[...] = 1.0 - p0


def _router(x, Wr, br):
    e0, e1, p0, p1 = pl.pallas_call(
        _router_body,
        out_shape=(
            jax.ShapeDtypeStruct((1, T), jnp.int32),
            jax.ShapeDtypeStruct((1, T), jnp.int32),
            jax.ShapeDtypeStruct((1, T), jnp.float32),
            jax.ShapeDtypeStruct((1, T), jnp.float32),
        ),
    )(x, Wr, br)
    return e0[0], e1[0], p0[0], p1[0]


def _gmm_body(te_ref, xs_ref, wg_ref, bg_ref, wu_ref, bu_ref, wd_ref, bd_ref,
              y_ref, acc_ref):
    h = pl.program_id(0)
    t = pl.program_id(1)

    @pl.when(te_ref[t] >= 0)
    def _():
        xb = xs_ref[...].astype(jnp.bfloat16)
        g = lax.dot_general(xb, wg_ref[0].astype(jnp.bfloat16),
                            (((1,), (1,)), ((), ())),
                            preferred_element_type=jnp.float32)
        g = g + bg_ref[0]
        g = g * jax.nn.sigmoid(g)
        u = lax.dot_general(xb, wu_ref[0].astype(jnp.bfloat16),
                            (((1,), (1,)), ((), ())),
                            preferred_element_type=jnp.float32)
        u = u + bu_ref[0]
        contrib = lax.dot_general((u * g).astype(jnp.bfloat16),
                                  wd_ref[0].astype(jnp.bfloat16),
                                  (((1,), (1,)), ((), ())),
                                  preferred_element_type=jnp.float32)
        rows = pl.ds(t * M, M)

        @pl.when(h == 0)
        def _():
            acc_ref[rows, :] = contrib + bd_ref[0]

        @pl.when((h > 0) & (h < NH - 1))
        def _():
            acc_ref[rows, :] = acc_ref[rows, :] + contrib

        @pl.when(h == NH - 1)
        def _():
            y_ref[...] = acc_ref[rows, :] + contrib


def _gmm(xs, te, Wg, bg, Wu, bu, Wd, bd):
    def we(te, t):
        return jnp.where(te[t] < 0, E - 1, te[t])

    grid_spec = pltpu.PrefetchScalarGridSpec(
        num_scalar_prefetch=1,
        grid=(NH, NT),
        in_specs=[
            pl.BlockSpec((M, D), lambda h, t, te: (t, 0)),
            pl.BlockSpec((1, HC, D), lambda h, t, te: (we(te, t), h, 0)),
            pl.BlockSpec((1, 1, HC),
                         lambda h, t, te: (we(te, t) * NH + h, 0, 0)),
            pl.BlockSpec((1, HC, D), lambda h, t, te: (we(te, t), h, 0)),
            pl.BlockSpec((1, 1, HC),
                         lambda h, t, te: (we(te, t) * NH + h, 0, 0)),
            pl.BlockSpec((1, D, HC), lambda h, t, te: (we(te, t), 0, h)),
            pl.BlockSpec((1, 1, D), lambda h, t, te: (we(te, t), 0, 0)),
        ],
        out_specs=pl.BlockSpec((M, D), lambda h, t, te: (t, 0)),
        scratch_shapes=[pltpu.VMEM((NP, D), jnp.float32)],
    )
    return pl.pallas_call(
        _gmm_body,
        grid_spec=grid_spec,
        out_shape=jax.ShapeDtypeStruct((NP, D), jnp.float32),
        compiler_params=pltpu.CompilerParams(
            dimension_semantics=("arbitrary", "arbitrary"),
        ),
    )(te, xs, Wg, bg.reshape(E * NH, 1, HC), Wu, bu.reshape(E * NH, 1, HC),
      Wd, bd.reshape(E, 1, D))


NW = 16
A = (2 * T) // NW
GR = 64


def _dispatch_body(e0_hbm, e1_hbm, x_hbm,
                   xs_hbm, pos0_hbm, pos1_hbm, te_hbm, cnts_hbm,
                   ea, dst, cnt1, allcnt, te_s, tid, did, rows, sem):
    cid = lax.axis_index("c")
    sid = lax.axis_index("s")
    iota16 = lax.broadcasted_iota(jnp.int32, (16,), 0)
    NC = A // 16

    @pl.when(cid == 0)
    def _():
        @pl.when(sid < NW // 2)
        def _():
            pltpu.sync_copy(e0_hbm.at[pl.ds(sid * A, A)], ea)

        @pl.when(sid >= NW // 2)
        def _():
            pltpu.sync_copy(e1_hbm.at[pl.ds((sid - NW // 2) * A, A)], ea)

        dn = lax.GatherDimensionNumbers(offset_dims=(),
                                        collapsed_slice_dims=(0,),
                                        start_index_map=(0,))

        def gat(v, idx):
            return lax.gather(v, idx[:, None], dn, (1,),
                              mode=lax.GatherScatterMode.PROMISE_IN_BOUNDS)

        rot_idx = [((iota16 - k) + 16) & 15 for k in (1, 2, 4, 8)]
        sh_idx = [jnp.maximum(iota16 - k, 0) for k in (1, 2, 4, 8)]
        sh_msk = [iota16 >= k for k in (1, 2, 4, 8)]

        def allsum(v):
            for idx in rot_idx:
                v = v + gat(v, idx)
            return v

        def iprefix(v):
            for idx, mk in zip(sh_idx, sh_msk):
                v = v + jnp.where(mk, gat(v, idx), 0)
            return v

        cv = jnp.zeros((16,), jnp.int32)
        for c in range(NC):
            ch = ea[pl.ds(c * 16, 16)]
            for e in range(E):
                pop = allsum(jnp.where(ch == e, 1, 0))
                cv = cv + jnp.where(iota16 == e, pop, 0)
        cnt1[...] = cv
        pltpu.sync_copy(cnt1, cnts_hbm.at[sid])
        plsc.subcore_barrier()
        pltpu.sync_copy(cnts_hbm, allcnt)

        tot = jnp.zeros((16,), jnp.int32)
        for wp in range(NW):
            tot = tot + allcnt[wp, :]
        a = tot + (M - 1)
        pv = (((a >> 6) * 43691) >> 17) * M
        csum = iprefix(pv)
        basev = csum - pv
        sv = basev
        for wp in range(NW):
            sv = sv + jnp.where(wp < sid, allcnt[wp, :], 0)

        for c in range(NC):
            ch = ea[pl.ds(c * 16, 16)]
            dv = gat(sv, ch)
            inc = jnp.zeros((16,), jnp.int32)
            for e in range(E):
                m = ch == e
                ip = iprefix(jnp.where(m, 1, 0))
                dv = dv + jnp.where(m, ip - 1, 0)
                cnt_e = jnp.full((16,), ip[15], jnp.int32)
                inc = inc + jnp.where(iota16 == e, cnt_e, 0)
            dst[pl.ds(c * 16, 16)] = dv
            sv = sv + inc

        @pl.when(sid < NW // 2)
        def _():
            pltpu.sync_copy(dst, pos0_hbm.at[pl.ds(sid * A, A)])

        @pl.when(sid >= NW // 2)
        def _():
            pltpu.sync_copy(dst, pos1_hbm.at[pl.ds((sid - NW // 2) * A, A)])

        tbase = jnp.where(sid < NW // 2, sid, sid - NW // 2) * A
        for r in range(A // GR):
            for v in range(GR // 16):
                tid[pl.ds(v * 16, 16)] = iota16 + (tbase + r * GR + v * 16)
                did[pl.ds(v * 16, 16)] = dst[pl.ds(r * GR + v * 16, 16)]
            pltpu.async_copy(x_hbm.at[tid], rows, sem).wait()
            pltpu.async_copy(rows, xs_hbm.at[did], sem).wait()

        @pl.when(sid == 0)
        def _():
            used = jnp.full((16,), basev[E], jnp.int32)
            for ch2 in range(2):
                bm = (iota16 + 16 * ch2) * M
                cntge = jnp.zeros((16,), jnp.int32)
                for e in range(E):
                    be = jnp.full((16,), basev[e], jnp.int32)
                    cntge = cntge + jnp.where(bm >= be, 1, 0)
                te_s[pl.ds(ch2 * 16, 16)] = jnp.where(bm >= used, -1,
                                                      cntge - 1)
            pltpu.sync_copy(te_s, te_hbm)


def _dispatch(e0, e1, x):
    mesh = plsc.VectorSubcoreMesh(core_axis_name="c", subcore_axis_name="s")
    f = functools.partial(
        pl.kernel,
        mesh=mesh,
        out_type=(
            jax.ShapeDtypeStruct((NP, D), jnp.float32),
            jax.ShapeDtypeStruct((T,), jnp.int32),
            jax.ShapeDtypeStruct((T,), jnp.int32),
            jax.ShapeDtypeStruct((32,), jnp.int32),
            jax.ShapeDtypeStruct((NW, 16), jnp.int32),
        ),
        scratch_types=[
            pltpu.VMEM((A,), jnp.int32),
            pltpu.VMEM((A,), jnp.int32),
            pltpu.VMEM((16,), jnp.int32),
            pltpu.VMEM((NW, 16), jnp.int32),
            pltpu.VMEM((32,), jnp.int32),
            pltpu.VMEM((GR,), jnp.int32),
            pltpu.VMEM((GR,), jnp.int32),
            pltpu.VMEM((GR, D), jnp.float32),
            pltpu.SemaphoreType.DMA,
        ],
    )(_dispatch_body)
    return f(e0, e1, x)


TW = T // 32


def _combine_body(ys_hbm, pos0_hbm, pos1_hbm, p0_hbm, p1_hbm, out_hbm,
                  i0b, i1b, p0b, p1b, r0, r1, sem):
    cid = lax.axis_index("c")
    sid = lax.axis_index("s")
    base = (cid * 16 + sid) * TW
    pltpu.sync_copy(pos0_hbm.at[pl.ds(base, TW)], i0b)
    pltpu.sync_copy(pos1_hbm.at[pl.ds(base, TW)], i1b)
    pltpu.sync_copy(p0_hbm.at[pl.ds(base, TW)], p0b.at[pl.ds(0, TW)])
    pltpu.sync_copy(p1_hbm.at[pl.ds(base, TW)], p1b.at[pl.ds(0, TW)])
    c0 = pltpu.async_copy(ys_hbm.at[i0b], r0, sem)
    c1 = pltpu.async_copy(ys_hbm.at[i1b], r1, sem)
    c0.wait()
    c1.wait()

    def tok(i, c):
        a0 = p0b[pl.ds(i, 16)][0]
        a1 = p1b[pl.ds(i, 16)][0]
        for v in range(D // 16):
            sl = pl.ds(v * 16, 16)
            r0[i, sl] = a0 * r0[i, sl] + a1 * r1[i, sl]
        return c

    lax.fori_loop(0, TW, tok, 0)
    pltpu.sync_copy(r0, out_hbm.at[pl.ds(base, TW), :])


def _combine(ys, pos0, pos1, p0, p1):
    mesh = plsc.VectorSubcoreMesh(core_axis_name="c", subcore_axis_name="s")
    f = functools.partial(
        pl.kernel,
        mesh=mesh,
        out_type=jax.ShapeDtypeStruct((T, D), jnp.float32),
        scratch_types=[
            pltpu.VMEM((TW,), jnp.int32),
            pltpu.VMEM((TW,), jnp.int32),
            pltpu.VMEM((TW + 16,), jnp.float32),
            pltpu.VMEM((TW + 16,), jnp.float32),
            pltpu.VMEM((TW, D), jnp.float32),
            pltpu.VMEM((TW, D), jnp.float32),
            pltpu.SemaphoreType.DMA,
        ],
    )(_combine_body)
    return f(ys, pos0, pos1, p0, p1)


def kernel(x, Wr, br, Wg, bg, Wu, bu, Wd, bd):
    e0, e1, p0, p1 = _router(x, Wr, br)
    xs, pos0, pos1, te, _ = _dispatch(e0, e1, x)
    ys = _gmm(xs, te, Wg, bg, Wu, bu, Wd, bd)
    return _combine(ys, pos0, pos1, p0, p1)


# speedup vs baseline: 1.0591x; 1.0326x over previous
import functools

import jax
import jax.numpy as jnp
from jax import lax
from jax.experimental import pallas as pl
from jax.experimental.pallas import tpu as pltpu
from jax.experimental.pallas import tpu_sc as plsc

T = 2048
D = 768
E = 8
K = 2
H = 3072

M = 192
HC = 1536
NP = 5760
NT = NP // M
NH = H // HC


def _router_body(x_ref, wr_ref, br_ref, e0_ref, e1_ref, p0_ref, p1_ref):
    lg = lax.dot_general(
        wr_ref[...], x_ref[...], (((1,), (1,)), ((), ())),
        preferred_element_type=jnp.float32,
    ) + br_ref[...][:, None]
    iota_e = lax.broadcasted_iota(jnp.int32, (E, T), 0)
    m0 = jnp.max(lg, axis=0, keepdims=True)
    i0 = jnp.min(jnp.where(lg == m0, iota_e, E), axis=0, keepdims=True)
    masked = jnp.where(iota_e == i0, -jnp.inf, lg)
    m1 = jnp.max(masked, axis=0, keepdims=True)
    i1 = jnp.min(jnp.where(masked == m1, iota_e, E), axis=0, keepdims=True)
    p0 = 1.0 / (1.0 + jnp.exp(m1 - m0))
    e0_ref[...] = i0
    e1_ref[...] = i1
    p0_ref[...] = p0
    p1_ref[...] = 1.0 - p0


def _router(x, Wr, br):
    e0, e1, p0, p1 = pl.pallas_call(
        _router_body,
        out_shape=(
            jax.ShapeDtypeStruct((1, T), jnp.int32),
            jax.ShapeDtypeStruct((1, T), jnp.int32),
            jax.ShapeDtypeStruct((1, T), jnp.float32),
            jax.ShapeDtypeStruct((1, T), jnp.float32),
        ),
    )(x, Wr, br)
    return e0[0], e1[0], p0[0], p1[0]


def _gmm_body(te_ref, xs_ref, wg_ref, bg_ref, wu_ref, bu_ref, wd_ref, bd_ref,
              y_ref, acc_ref):
    h = pl.program_id(0)
    t = pl.program_id(1)

    @pl.when(te_ref[t] >= 0)
    def _():
        xb = xs_ref[...].astype(jnp.bfloat16)
        g = lax.dot_general(xb, wg_ref[0].astype(jnp.bfloat16),
                            (((1,), (1,)), ((), ())),
                            preferred_element_type=jnp.float32)
        g = g + bg_ref[0]
        g = g * jax.nn.sigmoid(g)
        u = lax.dot_general(xb, wu_ref[0].astype(jnp.bfloat16),
                            (((1,), (1,)), ((), ())),
                            preferred_element_type=jnp.float32)
        u = u + bu_ref[0]
        contrib = lax.dot_general((u * g).astype(jnp.bfloat16),
                                  wd_ref[0].astype(jnp.bfloat16),
                                  (((1,), (1,)), ((), ())),
                                  preferred_element_type=jnp.float32)
        rows = pl.ds(t * M, M)

        @pl.when(h == 0)
        def _():
            acc_ref[rows, :] = contrib + bd_ref[0]

        @pl.when((h > 0) & (h < NH - 1))
        def _():
            acc_ref[rows, :] = acc_ref[rows, :] + contrib

        @pl.when(h == NH - 1)
        def _():
            y_ref[...] = acc_ref[rows, :] + contrib


def _gmm(xs, te, Wg, bg, Wu, bu, Wd, bd):
    def we(te, t):
        return jnp.where(te[t] < 0, E - 1, te[t])

    grid_spec = pltpu.PrefetchScalarGridSpec(
        num_scalar_prefetch=1,
        grid=(NH, NT),
        in_specs=[
            pl.BlockSpec((M, D), lambda h, t, te: (t, 0)),
            pl.BlockSpec((1, HC, D), lambda h, t, te: (we(te, t), h, 0)),
            pl.BlockSpec((1, 1, HC),
                         lambda h, t, te: (we(te, t) * NH + h, 0, 0)),
            pl.BlockSpec((1, HC, D), lambda h, t, te: (we(te, t), h, 0)),
            pl.BlockSpec((1, 1, HC),
                         lambda h, t, te: (we(te, t) * NH + h, 0, 0)),
            pl.BlockSpec((1, D, HC), lambda h, t, te: (we(te, t), 0, h)),
            pl.BlockSpec((1, 1, D), lambda h, t, te: (we(te, t), 0, 0)),
        ],
        out_specs=pl.BlockSpec((M, D), lambda h, t, te: (t, 0)),
        scratch_shapes=[pltpu.VMEM((NP, D), jnp.float32)],
    )
    return pl.pallas_call(
        _gmm_body,
        grid_spec=grid_spec,
        out_shape=jax.ShapeDtypeStruct((NP, D), jnp.float32),
        compiler_params=pltpu.CompilerParams(
            dimension_semantics=("arbitrary", "arbitrary"),
        ),
    )(te, xs, Wg, bg.reshape(E * NH, 1, HC), Wu, bu.reshape(E * NH, 1, HC),
      Wd, bd.reshape(E, 1, D))


NW = 16
A = (2 * T) // NW
GR = 64


def _dispatch_body(e0_hbm, e1_hbm, x_hbm,
                   xs_hbm, pos0_hbm, pos1_hbm, te_hbm, cnts_hbm,
                   ea, dst, cnt1, allcnt, te_s, tid, did, rows, sem):
    cid = lax.axis_index("c")
    sid = lax.axis_index("s")
    iota16 = lax.broadcasted_iota(jnp.int32, (16,), 0)
    NC = A // 16

    if True:
        @pl.when(sid < NW // 2)
        def _():
            pltpu.sync_copy(e0_hbm.at[pl.ds(sid * A, A)], ea)

        @pl.when(sid >= NW // 2)
        def _():
            pltpu.sync_copy(e1_hbm.at[pl.ds((sid - NW // 2) * A, A)], ea)

        dn = lax.GatherDimensionNumbers(offset_dims=(),
                                        collapsed_slice_dims=(0,),
                                        start_index_map=(0,))

        def gat(v, idx):
            return lax.gather(v, idx[:, None], dn, (1,),
                              mode=lax.GatherScatterMode.PROMISE_IN_BOUNDS)

        rot_idx = [((iota16 - k) + 16) & 15 for k in (1, 2, 4, 8)]
        sh_idx = [jnp.maximum(iota16 - k, 0) for k in (1, 2, 4, 8)]
        sh_msk = [iota16 >= k for k in (1, 2, 4, 8)]

        def allsum(v):
            for idx in rot_idx:
                v = v + gat(v, idx)
            return v

        def iprefix(v):
            for idx, mk in zip(sh_idx, sh_msk):
                v = v + jnp.where(mk, gat(v, idx), 0)
            return v

        cv = jnp.zeros((16,), jnp.int32)
        for c in range(NC):
            ch = ea[pl.ds(c * 16, 16)]
            for e in range(E):
                pop = allsum(jnp.where(ch == e, 1, 0))
                cv = cv + jnp.where(iota16 == e, pop, 0)
        cnt1[...] = cv
        pltpu.sync_copy(cnt1, cnts_hbm.at[cid * NW + sid])
        plsc.subcore_barrier()
        pltpu.sync_copy(cnts_hbm.at[pl.ds(cid * NW, NW), :], allcnt)

        tot = jnp.zeros((16,), jnp.int32)
        for wp in range(NW):
            tot = tot + allcnt[wp, :]
        a = tot + (M - 1)
        pv = (((a >> 6) * 43691) >> 17) * M
        csum = iprefix(pv)
        basev = csum - pv
        sv = basev
        for wp in range(NW):
            sv = sv + jnp.where(wp < sid, allcnt[wp, :], 0)

        for c in range(NC):
            ch = ea[pl.ds(c * 16, 16)]
            dv = gat(sv, ch)
            inc = jnp.zeros((16,), jnp.int32)
            for e in range(E):
                m = ch == e
                ip = iprefix(jnp.where(m, 1, 0))
                dv = dv + jnp.where(m, ip - 1, 0)
                cnt_e = jnp.full((16,), ip[15], jnp.int32)
                inc = inc + jnp.where(iota16 == e, cnt_e, 0)
            dst[pl.ds(c * 16, 16)] = dv
            sv = sv + inc

        @pl.when((cid == 0) & (sid < NW // 2))
        def _():
            pltpu.sync_copy(dst, pos0_hbm.at[pl.ds(sid * A, A)])

        @pl.when((cid == 0) & (sid >= NW // 2))
        def _():
            pltpu.sync_copy(dst, pos1_hbm.at[pl.ds((sid - NW // 2) * A, A)])

        tbase = jnp.where(sid < NW // 2, sid, sid - NW // 2) * A
        for r in range(A // GR):
            @pl.when(cid == r % 2)
            def _(r=r):
                for v in range(GR // 16):
                    tid[pl.ds(v * 16, 16)] = iota16 + (tbase + r * GR + v * 16)
                    did[pl.ds(v * 16, 16)] = dst[pl.ds(r * GR + v * 16, 16)]
                pltpu.async_copy(x_hbm.at[tid], rows, sem).wait()
                pltpu.async_copy(rows, xs_hbm.at[did], sem).wait()

        @pl.when((cid == 0) & (sid == 0))
        def _():
            used = jnp.full((16,), basev[E], jnp.int32)
            for ch2 in range(2):
                bm = (iota16 + 16 * ch2) * M
                cntge = jnp.zeros((16,), jnp.int32)
                for e in range(E):
                    be = jnp.full((16,), basev[e], jnp.int32)
                    cntge = cntge + jnp.where(bm >= be, 1, 0)
                te_s[pl.ds(ch2 * 16, 16)] = jnp.where(bm >= used, -1,
                                                      cntge - 1)
            pltpu.sync_copy(te_s, te_hbm)


def _dispatch(e0, e1, x):
    mesh = plsc.VectorSubcoreMesh(core_axis_name="c", subcore_axis_name="s")
    f = functools.partial(
        pl.kernel,
        mesh=mesh,
        out_type=(
            jax.ShapeDtypeStruct((NP, D), jnp.float32),
            jax.ShapeDtypeStruct((T,), jnp.int32),
            jax.ShapeDtypeStruct((T,), jnp.int32),
            jax.ShapeDtypeStruct((32,), jnp.int32),
            jax.ShapeDtypeStruct((2 * NW, 16), jnp.int32),
        ),
        scratch_types=[
            pltpu.VMEM((A,), jnp.int32),
            pltpu.VMEM((A,), jnp.int32),
            pltpu.VMEM((16,), jnp.int32),
            pltpu.VMEM((NW, 16), jnp.int32),
            pltpu.VMEM((32,), jnp.int32),
            pltpu.VMEM((GR,), jnp.int32),
            pltpu.VMEM((GR,), jnp.int32),
            pltpu.VMEM((GR, D), jnp.float32),
            pltpu.SemaphoreType.DMA,
        ],
    )(_dispatch_body)
    return f(e0, e1, x)


TW = T // 32


def _combine_body(ys_hbm, pos0_hbm, pos1_hbm, p0_hbm, p1_hbm, out_hbm,
                  i0b, i1b, p0b, p1b, r0, r1, sem):
    cid = lax.axis_index("c")
    sid = lax.axis_index("s")
    base = (cid * 16 + sid) * TW
    pltpu.sync_copy(pos0_hbm.at[pl.ds(base, TW)], i0b)
    pltpu.sync_copy(pos1_hbm.at[pl.ds(base, TW)], i1b)
    pltpu.sync_copy(p0_hbm.at[pl.ds(base, TW)], p0b.at[pl.ds(0, TW)])
    pltpu.sync_copy(p1_hbm.at[pl.ds(base, TW)], p1b.at[pl.ds(0, TW)])
    c0 = pltpu.async_copy(ys_hbm.at[i0b], r0, sem)
    c1 = pltpu.async_copy(ys_hbm.at[i1b], r1, sem)
    c0.wait()
    c1.wait()

    def tok(i, c):
        a0 = p0b[pl.ds(i, 16)][0]
        a1 = p1b[pl.ds(i, 16)][0]
        for v in range(D // 16):
            sl = pl.ds(v * 16, 16)
            r0[i, sl] = a0 * r0[i, sl] + a1 * r1[i, sl]
        return c

    lax.fori_loop(0, TW, tok, 0)
    pltpu.sync_copy(r0, out_hbm.at[pl.ds(base, TW), :])


def _combine(ys, pos0, pos1, p0, p1):
    mesh = plsc.VectorSubcoreMesh(core_axis_name="c", subcore_axis_name="s")
    f = functools.partial(
        pl.kernel,
        mesh=mesh,
        out_type=jax.ShapeDtypeStruct((T, D), jnp.float32),
        scratch_types=[
            pltpu.VMEM((TW,), jnp.int32),
            pltpu.VMEM((TW,), jnp.int32),
            pltpu.VMEM((TW + 16,), jnp.float32),
            pltpu.VMEM((TW + 16,), jnp.float32),
            pltpu.VMEM((TW, D), jnp.float32),
            pltpu.VMEM((TW, D), jnp.float32),
            pltpu.SemaphoreType.DMA,
        ],
    )(_combine_body)
    return f(ys, pos0, pos1, p0, p1)


def kernel(x, Wr, br, Wg, bg, Wu, bu, Wd, bd):
    e0, e1, p0, p1 = _router(x, Wr, br)
    xs, pos0, pos1, te, _ = _dispatch(e0, e1, x)
    ys = _gmm(xs, te, Wg, bg, Wu, bu, Wd, bd)
    return _combine(ys, pos0, pos1, p0, p1)
